# baseline (speedup 1.0000x reference)
import functools

import jax
import jax.numpy as jnp
from jax import lax
from jax.experimental import pallas as pl
from jax.experimental.pallas import tpu as pltpu
from jax.experimental.pallas import tpu_sc as plsc

BATCH = 16384
EMBED = 32
HIDDEN = 128
MLP_BLOCK = 2048

NUM_CORES = 2
NUM_SUBCORES = 16
NUM_WORKERS = NUM_CORES * NUM_SUBCORES
PER_WORKER = BATCH // NUM_WORKERS
CHUNK = 128


def _sc_gather(user_emb, anime_emb, user_ids, anime_ids):
    mesh = plsc.VectorSubcoreMesh(core_axis_name="c", subcore_axis_name="s")
    out_t = (jax.ShapeDtypeStruct((BATCH, EMBED), jnp.float32),
             jax.ShapeDtypeStruct((BATCH, EMBED), jnp.float32))

    @functools.partial(
        pl.kernel, mesh=mesh, out_type=out_t,
        compiler_params=pltpu.CompilerParams(use_tc_tiling_on_sc=False),
        scratch_types=[
            pltpu.VMEM((CHUNK,), jnp.int32),
            pltpu.VMEM((CHUNK,), jnp.int32),
            pltpu.VMEM((CHUNK, EMBED), jnp.float32),
            pltpu.VMEM((CHUNK, EMBED), jnp.float32),
            pltpu.SemaphoreType.DMA,
            pltpu.SemaphoreType.DMA,
        ],
    )
    def gather_kernel(ue_hbm, ae_hbm, ui_hbm, ai_hbm, uo_hbm, ao_hbm,
                      uix, aix, urows, arows, usem, asem):
        wid = lax.axis_index("s") * NUM_CORES + lax.axis_index("c")
        base = wid * PER_WORKER

        @pl.loop(0, PER_WORKER, step=CHUNK)
        def _(off):
            b = base + off
            pltpu.sync_copy(ui_hbm.at[pl.ds(b, CHUNK)], uix)
            pltpu.sync_copy(ai_hbm.at[pl.ds(b, CHUNK)], aix)
            cu = pltpu.async_copy(ue_hbm.at[uix], urows, usem)
            ca = pltpu.async_copy(ae_hbm.at[aix], arows, asem)
            cu.wait()
            ca.wait()
            pltpu.sync_copy(urows, uo_hbm.at[pl.ds(b, CHUNK)])
            pltpu.sync_copy(arows, ao_hbm.at[pl.ds(b, CHUNK)])

    return gather_kernel(user_emb, anime_emb, user_ids, anime_ids)


def _mlp_body(u_ref, a_ref, w1u_ref, w1a_ref, b1_ref, w2_ref, b2_ref, o_ref):
    h = jnp.dot(u_ref[...], w1u_ref[...], preferred_element_type=jnp.float32)
    h = h + jnp.dot(a_ref[...], w1a_ref[...],
                    preferred_element_type=jnp.float32)
    h = jnp.maximum(h + b1_ref[...], 0.0)
    o_ref[...] = jnp.sum(h * w2_ref[...], axis=1) + b2_ref[0, 0]


def _mlp(u, a, W1, b1, W2, b2):
    w1u = W1[:EMBED]
    w1a = W1[EMBED:]
    b1r = b1.reshape(1, HIDDEN)
    w2r = W2.reshape(1, HIDDEN)
    b2r = b2.reshape(1, 1)
    grid = (BATCH // MLP_BLOCK,)
    return pl.pallas_call(
        _mlp_body,
        grid=grid,
        in_specs=[
            pl.BlockSpec((MLP_BLOCK, EMBED), lambda i: (i, 0)),
            pl.BlockSpec((MLP_BLOCK, EMBED), lambda i: (i, 0)),
            pl.BlockSpec((EMBED, HIDDEN), lambda i: (0, 0)),
            pl.BlockSpec((EMBED, HIDDEN), lambda i: (0, 0)),
            pl.BlockSpec((1, HIDDEN), lambda i: (0, 0)),
            pl.BlockSpec((1, HIDDEN), lambda i: (0, 0)),
            pl.BlockSpec((1, 1), lambda i: (0, 0)),
        ],
        out_specs=pl.BlockSpec((MLP_BLOCK,), lambda i: (i,)),
        out_shape=jax.ShapeDtypeStruct((BATCH,), jnp.float32),
    )(u, a, w1u, w1a, b1r, w2r, b2r)


@jax.jit
def kernel(user_ids, anime_ids, user_emb, anime_emb, W1, b1, W2, b2):
    u, a = _sc_gather(user_emb, anime_emb, user_ids, anime_ids)
    return _mlp(u, a, W1, b1, W2, b2)
